# baseline (speedup 1.0000x reference)
import functools

import jax
import jax.numpy as jnp
from jax import lax
from jax.experimental import pallas as pl
from jax.experimental.pallas import tpu as pltpu
from jax.experimental.pallas import tpu_sc as plsc

_H = 64
_NW = 32
_CH = 320
_K = 2


@functools.cache
def _build(n):
    n_per_w = n // _NW
    nch = n_per_w // _CH
    mesh = plsc.VectorSubcoreMesh(core_axis_name="c", subcore_axis_name="s")

    @functools.partial(
        pl.kernel,
        out_type=jax.ShapeDtypeStruct((n, _H), jnp.float32),
        mesh=mesh,
        scratch_types=[
            pltpu.VMEM((n_per_w,), jnp.int32),
            pltpu.VMEM((_K, _CH, 128), jnp.float32),
            pltpu.SemaphoreType.DMA((_K,)),
            pltpu.SemaphoreType.DMA((_K,)),
        ],
        compiler_params=pltpu.CompilerParams(use_tc_tiling_on_sc=False),
    )
    def gather_kernel(idx_hbm, table_hbm, out_hbm, idx_v, bufs, gsem, ssem):
        wid = lax.axis_index("s") * 2 + lax.axis_index("c")
        base = wid * n_per_w
        pltpu.sync_copy(idx_hbm.at[pl.ds(base, n_per_w)], idx_v)

        def gather_desc(c, slot):
            return pltpu.make_async_copy(
                table_hbm.at[idx_v.at[pl.ds(c * _CH, _CH)]],
                bufs.at[slot], gsem.at[slot])

        def store_desc(c, slot):
            return pltpu.make_async_copy(
                bufs.at[slot, :, pl.ds(0, _H)],
                out_hbm.at[pl.ds(base + c * _CH, _CH)],
                ssem.at[slot])

        gather_desc(0, 0).start()

        def body(jj, carry):
            for b in range(_K):
                c = jj * _K + b
                gather_desc(c, b).wait()
                store_desc(c, b).start()

                @pl.when(c + 1 < nch)
                def _():
                    nb = (b + 1) % _K
                    @pl.when(c >= 1)
                    def _():
                        store_desc(c - 1, nb).wait()
                    gather_desc(c + 1, nb).start()

            return carry

        lax.fori_loop(0, nch // _K, body, 0)
        store_desc(nch - 2, (nch - 2) % _K).wait()
        store_desc(nch - 1, (nch - 1) % _K).wait()

    return gather_kernel


def kernel(token_ids, embed_table):
    b, s = token_ids.shape
    idx = token_ids.astype(jnp.int32).reshape(b * s)
    table128 = jnp.pad(embed_table, ((0, 0), (0, 128 - _H)))
    out = _build(b * s)(idx, table128)
    return out.reshape(b, s, _H)


# speedup vs baseline: 1.0997x; 1.0997x over previous
import functools

import jax
import jax.numpy as jnp
from jax import lax
from jax.experimental import pallas as pl
from jax.experimental.pallas import tpu as pltpu
from jax.experimental.pallas import tpu_sc as plsc

_H = 64
_NW = 32
_CH = 640
_K = 2


@functools.cache
def _build(n):
    n_per_w = n // _NW
    nch = n_per_w // _CH
    mesh = plsc.VectorSubcoreMesh(core_axis_name="c", subcore_axis_name="s")

    @functools.partial(
        pl.kernel,
        out_type=jax.ShapeDtypeStruct((n, _H), jnp.float32),
        mesh=mesh,
        scratch_types=[
            pltpu.VMEM((n_per_w,), jnp.int32),
            pltpu.VMEM((_K, _CH, _H), jnp.float32),
            pltpu.SemaphoreType.DMA((_K,)),
            pltpu.SemaphoreType.DMA((_K,)),
        ],
        compiler_params=pltpu.CompilerParams(use_tc_tiling_on_sc=False),
    )
    def gather_kernel(idx_hbm, table_hbm, out_hbm, idx_v, bufs, gsem, ssem):
        wid = lax.axis_index("s") * 2 + lax.axis_index("c")
        base = wid * n_per_w
        pltpu.sync_copy(idx_hbm.at[pl.ds(base, n_per_w)], idx_v)

        def gather_desc(c, slot):
            return pltpu.make_async_copy(
                table_hbm.at[idx_v.at[pl.ds(c * _CH, _CH)]],
                bufs.at[slot], gsem.at[slot])

        def store_desc(c, slot):
            return pltpu.make_async_copy(
                bufs.at[slot], out_hbm.at[pl.ds(base + c * _CH, _CH)],
                ssem.at[slot])

        gather_desc(0, 0).start()

        def body(jj, carry):
            for b in range(_K):
                c = jj * _K + b
                gather_desc(c, b).wait()
                store_desc(c, b).start()

                @pl.when(c + 1 < nch)
                def _():
                    nb = (b + 1) % _K
                    @pl.when(c >= 1)
                    def _():
                        store_desc(c - 1, nb).wait()
                    gather_desc(c + 1, nb).start()

            return carry

        lax.fori_loop(0, nch // _K, body, 0)
        store_desc(nch - 2, (nch - 2) % _K).wait()
        store_desc(nch - 1, (nch - 1) % _K).wait()

    return gather_kernel


def kernel(token_ids, embed_table):
    b, s = token_ids.shape
    idx = token_ids.astype(jnp.int32).reshape(b * s)
    out = _build(b * s)(idx, embed_table)
    return out.reshape(b, s, _H)
